# baseline (speedup 1.0000x reference)
import functools

import jax
import jax.numpy as jnp
from jax import lax
from jax.experimental import pallas as pl
from jax.experimental.pallas import tpu as pltpu
from jax.experimental.pallas import tpu_sc as plsc

_B = 16384
_D = 32
_NC = 2
_NS = 16
_NW = _NC * _NS
_BPW = _B // _NW
_LANES = 16


def _sc_body(uidx_hbm, iidx_hbm, uemb_hbm, iemb_hbm, ubias_hbm, ibias_hbm,
             bias_out_hbm, part_out_hbm,
             uidx_v, iidx_v, urows_v, irows_v, ub_v, ib_v, bs_v, acc_v, sem):
    wid = lax.axis_index("s") * _NC + lax.axis_index("c")
    base = wid * _BPW

    pltpu.sync_copy(uidx_hbm.at[pl.ds(base, _BPW)], uidx_v)
    pltpu.sync_copy(iidx_hbm.at[pl.ds(base, _BPW)], iidx_v)

    c1 = pltpu.async_copy(uemb_hbm.at[uidx_v], urows_v, sem)
    c2 = pltpu.async_copy(iemb_hbm.at[iidx_v], irows_v, sem)
    c3 = pltpu.async_copy(ubias_hbm.at[uidx_v], ub_v, sem)
    c4 = pltpu.async_copy(ibias_hbm.at[iidx_v], ib_v, sem)
    c1.wait()
    c2.wait()
    c3.wait()
    c4.wait()

    def bias_body(k, _):
        s = pl.ds(k * _LANES, _LANES)
        bs_v[s] = ub_v[s] + ib_v[s]
        return 0

    lax.fori_loop(0, _BPW // _LANES, bias_body, 0)
    pltpu.sync_copy(bs_v, bias_out_hbm.at[pl.ds(base, _BPW)])

    def dot_body(r, acc):
        u0 = urows_v[r, pl.ds(0, _LANES)]
        u1 = urows_v[r, pl.ds(_LANES, _LANES)]
        i0 = irows_v[r, pl.ds(0, _LANES)]
        i1 = irows_v[r, pl.ds(_LANES, _LANES)]
        return acc + u0 * i0 + u1 * i1

    acc = lax.fori_loop(0, _BPW, dot_body, jnp.zeros((_LANES,), jnp.float32))
    acc_v[...] = acc
    pltpu.sync_copy(acc_v, part_out_hbm.at[pl.ds(wid * _LANES, _LANES)])


@jax.jit
def _sc_call(uidx, iidx, uemb, iemb, ubias, ibias):
    mesh = plsc.VectorSubcoreMesh(core_axis_name="c", subcore_axis_name="s")
    return pl.kernel(
        _sc_body,
        out_type=[
            jax.ShapeDtypeStruct((_B,), jnp.float32),
            jax.ShapeDtypeStruct((_NW * _LANES,), jnp.float32),
        ],
        mesh=mesh,
        compiler_params=pltpu.CompilerParams(use_tc_tiling_on_sc=False),
        scratch_types=[
            pltpu.VMEM((_BPW,), jnp.int32),
            pltpu.VMEM((_BPW,), jnp.int32),
            pltpu.VMEM((_BPW, _D), jnp.float32),
            pltpu.VMEM((_BPW, _D), jnp.float32),
            pltpu.VMEM((_BPW,), jnp.float32),
            pltpu.VMEM((_BPW,), jnp.float32),
            pltpu.VMEM((_BPW,), jnp.float32),
            pltpu.VMEM((_LANES,), jnp.float32),
            pltpu.SemaphoreType.DMA,
        ],
    )(uidx, iidx, uemb, iemb, ubias, ibias)


def _tc_body(part_ref, bias_ref, out_ref):
    s = jnp.sum(part_ref[...])
    out_ref[...] = bias_ref[...] + s


@jax.jit
def _tc_call(partials, bias_sum):
    return pl.pallas_call(
        _tc_body,
        out_shape=jax.ShapeDtypeStruct(bias_sum.shape, jnp.float32),
    )(partials, bias_sum)


def kernel(inputs, user_emb, user_bias_tab, item_emb, item_bias_tab):
    uidx = inputs[:, 0]
    iidx = inputs[:, 1]
    ubias = user_bias_tab.reshape(-1)
    ibias = item_bias_tab.reshape(-1)
    bias_sum, partials = _sc_call(uidx, iidx, user_emb, item_emb, ubias, ibias)
    out = _tc_call(partials.reshape(4, 128), bias_sum.reshape(128, 128))
    return out.reshape(_B, 1)
